# baseline (speedup 1.0000x reference)
import functools

import jax
import jax.numpy as jnp
from jax import lax
from jax.experimental import pallas as pl
from jax.experimental.pallas import tpu as pltpu
from jax.experimental.pallas import tpu_sc as plsc

E = 8
SEQ = 2048
D = 768
DFF = 3072
K = 4096

BT = 256
BF = 512
NF = DFF // BF
NBLK = K // BT + E
NP = NBLK * BT


def _gelu_tanh(v):
    return 0.5 * v * (1.0 + jnp.tanh(jnp.sqrt(2.0 / jnp.pi) * (v + 0.044715 * v ** 3)))


def _scores_body(x_ref, gw_ref, bias_ref, out_ref):
    lg = lax.dot_general(gw_ref[...], x_ref[...], (((1,), (1,)), ((), ())),
                         preferred_element_type=jnp.float32)
    out_ref[...] = jax.nn.sigmoid(lg + bias_ref[...])


def _scores(x_flat, gate_weight, expert_bias):
    return pl.pallas_call(
        _scores_body,
        out_shape=jax.ShapeDtypeStruct((E, SEQ), jnp.float32),
    )(x_flat, gate_weight, expert_bias)


def _mlp_body(be_ref, xg_ref, w1_ref, b1_ref, w2_ref, b2_ref, wp_ref, y_ref):
    x_b = xg_ref[...]
    h = lax.dot_general(x_b, w1_ref[0], (((1,), (1,)), ((), ())),
                        preferred_element_type=jnp.float32)
    h = _gelu_tanh(h + b1_ref[0])
    part = lax.dot_general(h, w2_ref[0], (((1,), (1,)), ((), ())),
                           preferred_element_type=jnp.float32)
    w = wp_ref[0, 0]
    y_ref[...] = (part + b2_ref[0]) * w[:, None]


def _grouped_mlp(xg, W1, b1, W2, b2, w_pad, blk_exp):
    grid_spec = pltpu.PrefetchScalarGridSpec(
        num_scalar_prefetch=1,
        grid=(NBLK,),
        in_specs=[
            pl.BlockSpec((BT, D), lambda m, be: (m, 0)),
            pl.BlockSpec((1, DFF, D), lambda m, be: (be[m], 0, 0)),
            pl.BlockSpec((1, 1, DFF), lambda m, be: (be[m], 0, 0)),
            pl.BlockSpec((1, D, DFF), lambda m, be: (be[m], 0, 0)),
            pl.BlockSpec((1, 1, D), lambda m, be: (be[m], 0, 0)),
            pl.BlockSpec((1, 1, BT), lambda m, be: (m, 0, 0)),
        ],
        out_specs=pl.BlockSpec((BT, D), lambda m, be: (m, 0)),
    )
    return pl.pallas_call(
        _mlp_body,
        grid_spec=grid_spec,
        out_shape=jax.ShapeDtypeStruct((NP, D), jnp.float32),
    )(blk_exp, xg, W1, b1.reshape(E, 1, DFF), W2, b2.reshape(E, 1, D),
      w_pad.reshape(NBLK, 1, BT))


NTILES = 32
GRPT = NP // NTILES
GCH = 64
GNCH = GRPT // GCH


def _gather_body(x_hbm, tok_hbm, xg_hbm, idx_v, rows_v, sem):
    tile = lax.axis_index("c") * 16 + lax.axis_index("s")
    base = tile * GRPT
    for i in range(GNCH):
        pltpu.sync_copy(tok_hbm.at[pl.ds(base + i * GCH, GCH)], idx_v)
        pltpu.async_copy(x_hbm.at[idx_v], rows_v, sem).wait()
        pltpu.sync_copy(rows_v, xg_hbm.at[pl.ds(base + i * GCH, GCH), :])


def _sc_gather(x_flat, tok_pad):
    mesh = plsc.VectorSubcoreMesh(core_axis_name="c", subcore_axis_name="s")
    run = pl.kernel(
        _gather_body,
        out_type=jax.ShapeDtypeStruct((NP, D), jnp.float32),
        mesh=mesh,
        scratch_types=[
            pltpu.VMEM((GCH,), jnp.int32),
            pltpu.VMEM((GCH, D), jnp.float32),
            pltpu.SemaphoreType.DMA,
        ],
    )
    return run(x_flat, tok_pad)


TPT = SEQ // NTILES


def _combine_body(y_hbm, pt_hbm, out_hbm, idx_v, stage_v, acc_v, sem):
    tile = lax.axis_index("c") * 16 + lax.axis_index("s")
    t0 = tile * TPT
    pltpu.sync_copy(pt_hbm.at[pl.ds(t0, TPT)], idx_v)
    pltpu.async_copy(y_hbm.at[idx_v], acc_v, sem).wait()
    for j in range(1, E):
        pltpu.sync_copy(pt_hbm.at[pl.ds(j * SEQ + t0, TPT)], idx_v)
        pltpu.async_copy(y_hbm.at[idx_v], stage_v, sem).wait()

        def _add_row(r, _):
            for cch in range(D // 16):
                acc_v[r, pl.ds(cch * 16, 16)] = (
                    acc_v[r, pl.ds(cch * 16, 16)] + stage_v[r, pl.ds(cch * 16, 16)])
            return 0

        lax.fori_loop(0, TPT, _add_row, 0)
    pltpu.sync_copy(acc_v, out_hbm.at[pl.ds(t0, TPT), :])


def _sc_combine(y, pt):
    mesh = plsc.VectorSubcoreMesh(core_axis_name="c", subcore_axis_name="s")
    run = pl.kernel(
        _combine_body,
        out_type=jax.ShapeDtypeStruct((SEQ, D), jnp.float32),
        mesh=mesh,
        scratch_types=[
            pltpu.VMEM((TPT,), jnp.int32),
            pltpu.VMEM((TPT, D), jnp.float32),
            pltpu.VMEM((TPT, D), jnp.float32),
            pltpu.SemaphoreType.DMA,
        ],
    )
    return run(y, pt.reshape(-1))


def kernel(x, gate_weight, expert_bias, W1, b1, W2, b2):
    Bsz, seq, Dm = x.shape
    x_flat = x.reshape(-1, Dm)

    scores = _scores(x_flat, gate_weight, expert_bias)

    flat = scores.reshape(-1)
    thr = lax.top_k(flat, K)[0][-1]
    gt = flat > thr
    n_gt = gt.sum().astype(jnp.int32)
    eq = flat == thr
    eqrank = jnp.cumsum(eq.astype(jnp.int32))
    sel_flat = gt | (eq & (eqrank <= (K - n_gt)))
    sel2d = sel_flat.reshape(E, SEQ).astype(jnp.int32)

    counts = sel2d.sum(axis=1)
    rank_t = jnp.cumsum(sel2d, axis=1)
    nblk_e = (counts + BT - 1) // BT
    cnb_in = jnp.cumsum(nblk_e)
    blk_start = BT * (cnb_in - nblk_e)

    posmat = blk_start[:, None] + rank_t - 1
    slotmat = jnp.cumsum(sel2d, axis=0) - sel2d

    slots = jnp.arange(E, dtype=jnp.int32)[:, None, None]
    hit = (sel2d[None, :, :] == 1) & (slotmat[None, :, :] == slots)
    pt = (jnp.where(hit, posmat[None, :, :] + 1, 0)).sum(axis=1) - 1
    pt = jnp.where(pt < 0, NP - 1, pt).astype(jnp.int32)

    scat_pos = jnp.where(sel_flat, posmat.reshape(-1), NP)
    tmat = jnp.broadcast_to(jnp.arange(SEQ, dtype=jnp.int32)[None, :], (E, SEQ))
    tok_pad = jnp.zeros((NP + 1,), jnp.int32).at[scat_pos].set(tmat.reshape(-1),
                                                               mode='drop')[:NP]

    used = cnb_in[-1]
    bids = jnp.arange(NBLK, dtype=jnp.int32)
    blk_exp = jnp.searchsorted(cnb_in, bids, side='right').astype(jnp.int32)
    blk_exp = jnp.where(bids < used, blk_exp, 0)

    e_row = jnp.repeat(blk_exp, BT)
    w_raw = flat[e_row * SEQ + tok_pad]
    row_in_grp = jnp.arange(NP, dtype=jnp.int32) - jnp.repeat(blk_start[blk_exp], BT)
    w_pad = jnp.where(row_in_grp < jnp.repeat(counts[blk_exp], BT), w_raw, 0.0)

    xg = _sc_gather(x_flat, tok_pad)
    y = _grouped_mlp(xg, W1, b1, W2, b2, w_pad, blk_exp)
    out = _sc_combine(y, pt)

    token_each_expert = counts.astype(jnp.float32) / float(K)
    ones_like_mean = jnp.ones((E,), jnp.float32)
    return (out.reshape(Bsz, seq, Dm), token_each_expert, ones_like_mean)


# speedup vs baseline: 1.4984x; 1.4984x over previous
import functools

import jax
import jax.numpy as jnp
from jax import lax
from jax.experimental import pallas as pl
from jax.experimental.pallas import tpu as pltpu
from jax.experimental.pallas import tpu_sc as plsc

E = 8
SEQ = 2048
D = 768
DFF = 3072
K = 4096

BT = 256
BF = 512
NF = DFF // BF
NBLK = K // BT + E
NP = NBLK * BT


def _gelu_tanh(v):
    return 0.5 * v * (1.0 + jnp.tanh(jnp.sqrt(2.0 / jnp.pi) * (v + 0.044715 * v ** 3)))


def _scores_body(x_ref, gw_ref, bias_ref, out_ref):
    lg = lax.dot_general(gw_ref[...], x_ref[...], (((1,), (1,)), ((), ())),
                         preferred_element_type=jnp.float32)
    out_ref[...] = jax.nn.sigmoid(lg + bias_ref[...])


def _scores(x_flat, gate_weight, expert_bias):
    return pl.pallas_call(
        _scores_body,
        out_shape=jax.ShapeDtypeStruct((E, SEQ), jnp.float32),
    )(x_flat, gate_weight, expert_bias)


def _mlp_body(be_ref, xg_ref, w1_ref, b1_ref, w2_ref, b2_ref, wp_ref, y_ref):
    m = pl.program_id(0)
    used = be_ref[NBLK]

    @pl.when(m < used)
    def _():
        x_b = xg_ref[...]
        h = lax.dot_general(x_b, w1_ref[0], (((1,), (1,)), ((), ())),
                            preferred_element_type=jnp.float32)
        h = _gelu_tanh(h + b1_ref[0])
        part = lax.dot_general(h, w2_ref[0], (((1,), (1,)), ((), ())),
                               preferred_element_type=jnp.float32)
        w = wp_ref[0, 0]
        y_ref[...] = (part + b2_ref[0]) * w[:, None]

    @pl.when(m >= used)
    def _():
        y_ref[...] = jnp.zeros_like(y_ref)


def _grouped_mlp(xg, W1, b1, W2, b2, w_pad, blk_exp_ext):
    grid_spec = pltpu.PrefetchScalarGridSpec(
        num_scalar_prefetch=1,
        grid=(NBLK,),
        in_specs=[
            pl.BlockSpec((BT, D), lambda m, be: (m, 0)),
            pl.BlockSpec((1, DFF, D), lambda m, be: (be[m], 0, 0)),
            pl.BlockSpec((1, 1, DFF), lambda m, be: (be[m], 0, 0)),
            pl.BlockSpec((1, D, DFF), lambda m, be: (be[m], 0, 0)),
            pl.BlockSpec((1, 1, D), lambda m, be: (be[m], 0, 0)),
            pl.BlockSpec((1, 1, BT), lambda m, be: (m, 0, 0)),
        ],
        out_specs=pl.BlockSpec((BT, D), lambda m, be: (m, 0)),
    )
    return pl.pallas_call(
        _mlp_body,
        grid_spec=grid_spec,
        out_shape=jax.ShapeDtypeStruct((NP, D), jnp.float32),
    )(blk_exp_ext, xg, W1, b1.reshape(E, 1, DFF), W2, b2.reshape(E, 1, D),
      w_pad.reshape(NBLK, 1, BT))


NTILES = 32
GRPT = NP // NTILES
GCH = 64
GNCH = GRPT // GCH


def _gather_body(x_hbm, tok_hbm, flags_hbm, xg_hbm, idx_v, rows_v, fl_v, sem):
    tile = lax.axis_index("c") * 16 + lax.axis_index("s")
    base = tile * GRPT
    pltpu.sync_copy(flags_hbm.at[pl.ds(tile * 16, 16)], fl_v)
    fl = fl_v[...]
    for i in range(GNCH):
        fi = fl[i]

        @pl.when(fi > 0)
        def _():
            pltpu.sync_copy(tok_hbm.at[pl.ds(base + i * GCH, GCH)], idx_v)
            pltpu.async_copy(x_hbm.at[idx_v], rows_v, sem).wait()
            pltpu.sync_copy(rows_v, xg_hbm.at[pl.ds(base + i * GCH, GCH), :])


def _sc_gather(x_flat, tok_pad, gflags):
    mesh = plsc.VectorSubcoreMesh(core_axis_name="c", subcore_axis_name="s")
    run = pl.kernel(
        _gather_body,
        out_type=jax.ShapeDtypeStruct((NP, D), jnp.float32),
        mesh=mesh,
        scratch_types=[
            pltpu.VMEM((GCH,), jnp.int32),
            pltpu.VMEM((GCH, D), jnp.float32),
            pltpu.VMEM((16,), jnp.int32),
            pltpu.SemaphoreType.DMA,
        ],
    )
    return run(x_flat, tok_pad, gflags.reshape(-1))


TPT = SEQ // NTILES


SUB = 32
NSUB = TPT // SUB


def _combine_body(y_hbm, pt_hbm, flags_hbm, out_hbm, idx_v, stage_v, acc_v, fl_v,
                  sem, osem):
    tile = lax.axis_index("c") * 16 + lax.axis_index("s")
    t0 = tile * TPT
    pbase = tile * (E * TPT)
    pltpu.sync_copy(flags_hbm.at[pl.ds(tile * 16, 16)], fl_v)
    fl = fl_v[...]
    for sub in range(NSUB):
        if sub > 0:
            pltpu.make_async_copy(
                acc_v, out_hbm.at[pl.ds(t0 + (sub - 1) * SUB, SUB), :],
                osem).wait()
        pltpu.sync_copy(pt_hbm.at[pl.ds(pbase + sub * SUB, SUB)], idx_v)
        pltpu.async_copy(y_hbm.at[idx_v], acc_v, sem).wait()
        for j in range(1, E):
            fj = fl[j]

            @pl.when(fj > 0)
            def _():
                pltpu.sync_copy(
                    pt_hbm.at[pl.ds(pbase + j * TPT + sub * SUB, SUB)], idx_v)
                pltpu.async_copy(y_hbm.at[idx_v], stage_v, sem).wait()

                def _add_row(r, _):
                    for cch in range(D // 16):
                        plsc.addupdate(acc_v.at[r, pl.ds(cch * 16, 16)],
                                       stage_v[r, pl.ds(cch * 16, 16)])
                    return 0

                lax.fori_loop(0, SUB, _add_row, 0)
        pltpu.make_async_copy(
            acc_v, out_hbm.at[pl.ds(t0 + sub * SUB, SUB), :], osem).start()
    pltpu.make_async_copy(
        acc_v, out_hbm.at[pl.ds(t0 + (NSUB - 1) * SUB, SUB), :], osem).wait()


def _sc_combine(y, pt_t, cflags):
    mesh = plsc.VectorSubcoreMesh(core_axis_name="c", subcore_axis_name="s")
    run = pl.kernel(
        _combine_body,
        out_type=jax.ShapeDtypeStruct((SEQ, D), jnp.float32),
        mesh=mesh,
        scratch_types=[
            pltpu.VMEM((SUB,), jnp.int32),
            pltpu.VMEM((SUB, D), jnp.float32),
            pltpu.VMEM((SUB, D), jnp.float32),
            pltpu.VMEM((16,), jnp.int32),
            pltpu.SemaphoreType.DMA,
            pltpu.SemaphoreType.DMA,
        ],
    )
    return run(y, pt_t.reshape(-1), cflags.reshape(-1))


def kernel(x, gate_weight, expert_bias, W1, b1, W2, b2):
    Bsz, seq, Dm = x.shape
    x_flat = x.reshape(-1, Dm)

    scores = _scores(x_flat, gate_weight, expert_bias)

    flat = scores.reshape(-1)
    thr = lax.top_k(flat, K)[0][-1]
    gt = flat > thr
    n_gt = gt.sum().astype(jnp.int32)
    eq = flat == thr
    eqrank = jnp.cumsum(eq.astype(jnp.int32))
    sel_flat = gt | (eq & (eqrank <= (K - n_gt)))
    sel2d = sel_flat.reshape(E, SEQ).astype(jnp.int32)

    counts = sel2d.sum(axis=1)
    rank_t = jnp.cumsum(sel2d, axis=1)
    nblk_e = (counts + BT - 1) // BT
    cnb_in = jnp.cumsum(nblk_e)
    blk_start = BT * (cnb_in - nblk_e)

    posmat = blk_start[:, None] + rank_t - 1
    slotmat = jnp.cumsum(sel2d, axis=0) - sel2d

    slots = jnp.arange(E, dtype=jnp.int32)[:, None, None]
    hit = (sel2d[None, :, :] == 1) & (slotmat[None, :, :] == slots)
    pt = (jnp.where(hit, posmat[None, :, :] + 1, 0)).sum(axis=1) - 1
    pt = jnp.where(pt < 0, NP - 1, pt).astype(jnp.int32)

    scat_pos = jnp.where(sel_flat, posmat.reshape(-1), NP)
    tmat = jnp.broadcast_to(jnp.arange(SEQ, dtype=jnp.int32)[None, :], (E, SEQ))
    tok_pad = jnp.zeros((NP + 1,), jnp.int32).at[scat_pos].set(tmat.reshape(-1),
                                                               mode='drop')[:NP]

    used = cnb_in[-1]
    bids = jnp.arange(NBLK, dtype=jnp.int32)
    blk_exp = jnp.searchsorted(cnb_in, bids, side='right').astype(jnp.int32)
    e_last = jnp.max(jnp.where(bids < used, blk_exp, -1))
    blk_exp = jnp.where(bids < used, blk_exp, e_last)

    e_row = jnp.repeat(blk_exp, BT)
    w_raw = flat[e_row * SEQ + tok_pad]
    row_in_grp = jnp.arange(NP, dtype=jnp.int32) - jnp.repeat(blk_start[blk_exp], BT)
    w_pad = jnp.where(row_in_grp < jnp.repeat(counts[blk_exp], BT), w_raw, 0.0)

    row0 = (jnp.arange(NTILES, dtype=jnp.int32) * GRPT)[:, None] \
        + (jnp.arange(16, dtype=jnp.int32) * GCH)[None, :]
    gflags = ((row0 < used * BT)
              & (jnp.arange(16, dtype=jnp.int32)[None, :] < GNCH)).astype(jnp.int32)

    pt_t = pt.reshape(E, NTILES, TPT).transpose(1, 0, 2)
    cflags = jnp.zeros((NTILES, 16), jnp.int32).at[:, :E].set(
        (pt_t != NP - 1).any(axis=2).astype(jnp.int32))

    blk_exp_ext = jnp.concatenate([blk_exp, used[None]])

    xg = _sc_gather(x_flat, tok_pad, gflags)
    y = _grouped_mlp(xg, W1, b1, W2, b2, w_pad, blk_exp_ext)
    out = _sc_combine(y, pt_t, cflags)

    token_each_expert = counts.astype(jnp.float32) / float(K)
    ones_like_mean = jnp.ones((E,), jnp.float32)
    return (out.reshape(Bsz, seq, Dm), token_each_expert, ones_like_mean)


# speedup vs baseline: 2.8334x; 1.8909x over previous
import functools

import jax
import jax.numpy as jnp
from jax import lax
from jax.experimental import pallas as pl
from jax.experimental.pallas import tpu as pltpu
from jax.experimental.pallas import tpu_sc as plsc

E = 8
SEQ = 2048
D = 768
DFF = 3072
K = 4096

BT = 256
BF = 512
NF = DFF // BF
NBLK = K // BT + E
NP = NBLK * BT


def _gelu_tanh(v):
    return 0.5 * v * (1.0 + jnp.tanh(jnp.sqrt(2.0 / jnp.pi) * (v + 0.044715 * v ** 3)))


def _scores_body(x_ref, gw_ref, bias_ref, out_ref):
    lg = lax.dot_general(gw_ref[...], x_ref[...], (((1,), (1,)), ((), ())),
                         preferred_element_type=jnp.float32)
    out_ref[...] = jax.nn.sigmoid(lg + bias_ref[...])


def _scores(x_flat, gate_weight, expert_bias):
    return pl.pallas_call(
        _scores_body,
        out_shape=jax.ShapeDtypeStruct((E, SEQ), jnp.float32),
    )(x_flat, gate_weight, expert_bias)


def _mlp_body(be_ref, xg_ref, w1_ref, b1_ref, w2_ref, b2_ref, wp_ref, tok_ref,
              out_ref):
    m = pl.program_id(0)
    used = be_ref[NBLK]

    @pl.when(m == 0)
    def _():
        out_ref[...] = jnp.zeros_like(out_ref)

    @pl.when(m < used)
    def _():
        x_b = xg_ref[...]
        h = lax.dot_general(x_b, w1_ref[0], (((1,), (1,)), ((), ())),
                            preferred_element_type=jnp.float32)
        h = _gelu_tanh(h + b1_ref[0])
        part = lax.dot_general(h, w2_ref[0], (((1,), (1,)), ((), ())),
                               preferred_element_type=jnp.float32)
        w = wp_ref[0, 0]
        y = (part + b2_ref[0]) * w[:, None]
        toks = tok_ref[0, 0]
        t_iota = lax.broadcasted_iota(jnp.int32, (BT, SEQ), 1)
        onehot = (toks[:, None] == t_iota).astype(jnp.float32)
        out_ref[...] += lax.dot_general(
            onehot, y, (((0,), (0,)), ((), ())),
            preferred_element_type=jnp.float32)


def _grouped_mlp(xg, W1, b1, W2, b2, w_pad, tok_pad, blk_exp_ext):
    grid_spec = pltpu.PrefetchScalarGridSpec(
        num_scalar_prefetch=1,
        grid=(NBLK,),
        in_specs=[
            pl.BlockSpec((BT, D), lambda m, be: (m, 0)),
            pl.BlockSpec((1, DFF, D), lambda m, be: (be[m], 0, 0)),
            pl.BlockSpec((1, 1, DFF), lambda m, be: (be[m], 0, 0)),
            pl.BlockSpec((1, D, DFF), lambda m, be: (be[m], 0, 0)),
            pl.BlockSpec((1, 1, D), lambda m, be: (be[m], 0, 0)),
            pl.BlockSpec((1, 1, BT), lambda m, be: (m, 0, 0)),
            pl.BlockSpec((1, 1, BT), lambda m, be: (m, 0, 0)),
        ],
        out_specs=pl.BlockSpec((SEQ, D), lambda m, be: (0, 0)),
    )
    return pl.pallas_call(
        _mlp_body,
        grid_spec=grid_spec,
        out_shape=jax.ShapeDtypeStruct((SEQ, D), jnp.float32),
    )(blk_exp_ext, xg, W1, b1.reshape(E, 1, DFF), W2, b2.reshape(E, 1, D),
      w_pad.reshape(NBLK, 1, BT), tok_pad.reshape(NBLK, 1, BT))


NTILES = 32
GRPT = NP // NTILES
GCH = 64
GNCH = GRPT // GCH


def _gather_body(x_hbm, tok_hbm, flags_hbm, xg_hbm, idx_v, rows_v, fl_v, sem):
    tile = lax.axis_index("c") * 16 + lax.axis_index("s")
    base = tile * GRPT
    pltpu.sync_copy(flags_hbm.at[pl.ds(tile * 16, 16)], fl_v)
    fl = fl_v[...]
    for i in range(GNCH):
        fi = fl[i]

        @pl.when(fi > 0)
        def _():
            pltpu.sync_copy(tok_hbm.at[pl.ds(base + i * GCH, GCH)], idx_v)
            pltpu.async_copy(x_hbm.at[idx_v], rows_v, sem).wait()
            pltpu.sync_copy(rows_v, xg_hbm.at[pl.ds(base + i * GCH, GCH), :])


def _sc_gather(x_flat, tok_pad, gflags):
    mesh = plsc.VectorSubcoreMesh(core_axis_name="c", subcore_axis_name="s")
    run = pl.kernel(
        _gather_body,
        out_type=jax.ShapeDtypeStruct((NP, D), jnp.float32),
        mesh=mesh,
        scratch_types=[
            pltpu.VMEM((GCH,), jnp.int32),
            pltpu.VMEM((GCH, D), jnp.float32),
            pltpu.VMEM((16,), jnp.int32),
            pltpu.SemaphoreType.DMA,
        ],
    )
    return run(x_flat, tok_pad, gflags.reshape(-1))


TPT = SEQ // NTILES


SUB = 32
NSUB = TPT // SUB


def _combine_body(y_hbm, pt_hbm, flags_hbm, out_hbm, idx_v, stage_v, acc_v, fl_v,
                  sem, osem):
    tile = lax.axis_index("c") * 16 + lax.axis_index("s")
    t0 = tile * TPT
    pbase = tile * (E * TPT)
    pltpu.sync_copy(flags_hbm.at[pl.ds(tile * 16, 16)], fl_v)
    fl = fl_v[...]
    for sub in range(NSUB):
        if sub > 0:
            pltpu.make_async_copy(
                acc_v, out_hbm.at[pl.ds(t0 + (sub - 1) * SUB, SUB), :],
                osem).wait()
        pltpu.sync_copy(pt_hbm.at[pl.ds(pbase + sub * SUB, SUB)], idx_v)
        pltpu.async_copy(y_hbm.at[idx_v], acc_v, sem).wait()
        for j in range(1, E):
            fj = fl[j]

            @pl.when(fj > 0)
            def _():
                pltpu.sync_copy(
                    pt_hbm.at[pl.ds(pbase + j * TPT + sub * SUB, SUB)], idx_v)
                pltpu.async_copy(y_hbm.at[idx_v], stage_v, sem).wait()

                def _add_row(r, _):
                    for cch in range(D // 16):
                        plsc.addupdate(acc_v.at[r, pl.ds(cch * 16, 16)],
                                       stage_v[r, pl.ds(cch * 16, 16)])
                    return 0

                lax.fori_loop(0, SUB, _add_row, 0)
        pltpu.make_async_copy(
            acc_v, out_hbm.at[pl.ds(t0 + sub * SUB, SUB), :], osem).start()
    pltpu.make_async_copy(
        acc_v, out_hbm.at[pl.ds(t0 + (NSUB - 1) * SUB, SUB), :], osem).wait()


def _sc_combine(y, pt_t, cflags):
    mesh = plsc.VectorSubcoreMesh(core_axis_name="c", subcore_axis_name="s")
    run = pl.kernel(
        _combine_body,
        out_type=jax.ShapeDtypeStruct((SEQ, D), jnp.float32),
        mesh=mesh,
        scratch_types=[
            pltpu.VMEM((SUB,), jnp.int32),
            pltpu.VMEM((SUB, D), jnp.float32),
            pltpu.VMEM((SUB, D), jnp.float32),
            pltpu.VMEM((16,), jnp.int32),
            pltpu.SemaphoreType.DMA,
            pltpu.SemaphoreType.DMA,
        ],
    )
    return run(y, pt_t.reshape(-1), cflags.reshape(-1))


def kernel(x, gate_weight, expert_bias, W1, b1, W2, b2):
    Bsz, seq, Dm = x.shape
    x_flat = x.reshape(-1, Dm)

    scores = _scores(x_flat, gate_weight, expert_bias)

    flat = scores.reshape(-1)
    thr = lax.top_k(flat, K)[0][-1]
    gt = flat > thr
    n_gt = gt.sum().astype(jnp.int32)
    eq = flat == thr
    eqrank = jnp.cumsum(eq.astype(jnp.int32))
    sel_flat = gt | (eq & (eqrank <= (K - n_gt)))
    sel2d = sel_flat.reshape(E, SEQ).astype(jnp.int32)

    counts = sel2d.sum(axis=1)
    rank_t = jnp.cumsum(sel2d, axis=1)
    nblk_e = (counts + BT - 1) // BT
    cnb_in = jnp.cumsum(nblk_e)
    blk_start = BT * (cnb_in - nblk_e)

    posmat = blk_start[:, None] + rank_t - 1

    scat_pos = jnp.where(sel_flat, posmat.reshape(-1), NP)
    tmat = jnp.broadcast_to(jnp.arange(SEQ, dtype=jnp.int32)[None, :], (E, SEQ))
    tok_pad = jnp.zeros((NP + 1,), jnp.int32).at[scat_pos].set(tmat.reshape(-1),
                                                               mode='drop')[:NP]

    used = cnb_in[-1]
    bids = jnp.arange(NBLK, dtype=jnp.int32)
    blk_exp = jnp.searchsorted(cnb_in, bids, side='right').astype(jnp.int32)
    e_last = jnp.max(jnp.where(bids < used, blk_exp, -1))
    blk_exp = jnp.where(bids < used, blk_exp, e_last)

    e_row = jnp.repeat(blk_exp, BT)
    w_raw = flat[e_row * SEQ + tok_pad]
    row_in_grp = jnp.arange(NP, dtype=jnp.int32) - jnp.repeat(blk_start[blk_exp], BT)
    w_pad = jnp.where(row_in_grp < jnp.repeat(counts[blk_exp], BT), w_raw, 0.0)

    row0 = (jnp.arange(NTILES, dtype=jnp.int32) * GRPT)[:, None] \
        + (jnp.arange(16, dtype=jnp.int32) * GCH)[None, :]
    gflags = ((row0 < used * BT)
              & (jnp.arange(16, dtype=jnp.int32)[None, :] < GNCH)).astype(jnp.int32)

    blk_exp_ext = jnp.concatenate([blk_exp, used[None]])

    xg = _sc_gather(x_flat, tok_pad, gflags)
    out = _grouped_mlp(xg, W1, b1, W2, b2, w_pad, tok_pad, blk_exp_ext)

    token_each_expert = counts.astype(jnp.float32) / float(K)
    ones_like_mean = jnp.ones((E,), jnp.float32)
    return (out.reshape(Bsz, seq, Dm), token_each_expert, ones_like_mean)
